# Initial kernel scaffold; baseline (speedup 1.0000x reference)
import jax
import jax.numpy as jnp
from jax.experimental import pallas as pl


def kernel(x, y1, edge_index, W1, b1, W2, b2, W3, b3):
    raise NotImplementedError("write your pallas kernel here")


# speedup vs baseline: 33.9351x; 33.9351x over previous
import functools

import jax
import jax.numpy as jnp
from jax import lax
from jax.experimental import pallas as pl
from jax.experimental.pallas import tpu as pltpu
from jax.experimental.pallas import tpu_sc as plsc

NC = 2
NS = 16
L = 16

CH = 128
K = 8
GROUP = CH * K
BUFR = 448

F32 = jnp.float32


def _mesh():
    return plsc.VectorSubcoreMesh(
        core_axis_name="c", subcore_axis_name="s", num_cores=NC, num_subcores=NS
    )


def _make_deg_kernel(n_pad, e_pad):
    e_half = e_pad // NC
    rows_per_tile_e = e_half // NS // CH
    n_groups = e_half // NS // GROUP
    rpt = n_pad // NS

    def body(dst2d, out, idx_v, ones_v, buf_v, acc, sem):
        c = lax.axis_index("c")
        s = lax.axis_index("s")
        row0 = s * rpt

        @pl.loop(0, rpt // L)
        def _zero(i):
            buf_v[pl.ds(i * L, L)] = jnp.zeros((L,), F32)

        @pl.loop(0, K)
        def _ones(j):
            ones_v[pl.ds(j * L, L)] = jnp.ones((L,), F32)

        pltpu.sync_copy(buf_v, acc.at[pl.ds(row0, rpt)])
        plsc.subcore_barrier()

        tile_erow0 = c * (e_half // CH) + s * rows_per_tile_e

        @pl.loop(0, n_groups)
        def _grp(g):
            pltpu.sync_copy(dst2d.at[pl.ds(tile_erow0 + g * K, K)], idx_v)
            descs = [
                pltpu.async_copy(ones_v, acc.at[idx_v.at[j]], sem, add=True)
                for j in range(K)
            ]
            for d in descs:
                d.wait()

        plsc.subcore_barrier()
        pltpu.sync_copy(acc.at[pl.ds(row0, rpt)], buf_v)
        pltpu.sync_copy(buf_v, out.at[c, pl.ds(row0, rpt)])

    return pl.kernel(
        body,
        out_type=jax.ShapeDtypeStruct((NC, n_pad), F32),
        mesh=_mesh(),
        compiler_params=pltpu.CompilerParams(use_tc_tiling_on_sc=False),
        scratch_types=[
            pltpu.VMEM((K, CH), jnp.int32),
            pltpu.VMEM((CH,), F32),
            pltpu.VMEM((rpt,), F32),
            pltpu.VMEM_SHARED((n_pad,), F32),
            pltpu.SemaphoreType.DMA,
        ],
    )


def _make_agg_kernel(n_pad, e_pad, n_tables):
    split_edges = n_tables == 1
    e_per_sc = e_pad // NC if split_edges else e_pad
    rows_per_tile_e = e_per_sc // NS // CH
    n_groups = e_per_sc // NS // GROUP
    rpt = n_pad // NS

    def body(table, src2d, dst2d, out, idxs, idxd, rows, buf, acc, semg, sems):
        c = lax.axis_index("c")
        s = lax.axis_index("s")
        row0 = s * rpt

        @pl.loop(0, BUFR)
        def _zero(i):
            buf[i, :] = jnp.zeros((L,), F32)

        @pl.loop(0, rpt // BUFR)
        def _zacc(i):
            pltpu.sync_copy(buf, acc.at[pl.ds(row0 + i * BUFR, BUFR)])

        plsc.subcore_barrier()

        if split_edges:
            tile_erow0 = c * (e_per_sc // CH) + s * rows_per_tile_e
        else:
            tile_erow0 = s * rows_per_tile_e

        @pl.loop(0, n_groups)
        def _grp(g):
            er = tile_erow0 + g * K
            pltpu.sync_copy(src2d.at[pl.ds(er, K)], idxs)
            pltpu.sync_copy(dst2d.at[pl.ds(er, K)], idxd)
            if not split_edges:
                off = c * n_pad
                for j in range(K):
                    for t in range(CH // L):
                        idxs[j, pl.ds(t * L, L)] = idxs[j, pl.ds(t * L, L)] + off
            gd = [
                pltpu.async_copy(table.at[idxs.at[j]], rows.at[j], semg)
                for j in range(K)
            ]
            for d in gd:
                d.wait()
            sd = [
                pltpu.async_copy(rows.at[j], acc.at[idxd.at[j]], sems, add=True)
                for j in range(K)
            ]
            for d in sd:
                d.wait()

        plsc.subcore_barrier()

        @pl.loop(0, rpt // BUFR)
        def _wr(i):
            pltpu.sync_copy(acc.at[pl.ds(row0 + i * BUFR, BUFR)], buf)
            pltpu.sync_copy(buf, out.at[c, pl.ds(row0 + i * BUFR, BUFR)])

    return pl.kernel(
        body,
        out_type=jax.ShapeDtypeStruct((NC, n_pad, L), F32),
        mesh=_mesh(),
        compiler_params=pltpu.CompilerParams(use_tc_tiling_on_sc=False),
        scratch_types=[
            pltpu.VMEM((K, CH), jnp.int32),
            pltpu.VMEM((K, CH), jnp.int32),
            pltpu.VMEM((K, CH, L), F32),
            pltpu.VMEM((BUFR, L), F32),
            pltpu.VMEM_SHARED((n_pad, L), F32),
            pltpu.SemaphoreType.DMA,
            pltpu.SemaphoreType.DMA,
        ],
    )


def _tc_dinv(d_parts):
    def body(d_ref, o_ref):
        deg = d_ref[0] + d_ref[1] + 1.0
        o_ref[...] = lax.rsqrt(deg)

    r = d_parts.shape[1]
    return pl.pallas_call(
        body,
        out_shape=jax.ShapeDtypeStruct((r, 128), F32),
    )(d_parts)


def _tc_scale(h_pad, dinv16):
    n_pad = h_pad.shape[0]
    bn = 2048
    grid = (n_pad // bn,)

    def body(h_ref, d_ref, o_ref):
        o_ref[...] = h_ref[...] * d_ref[...]

    return pl.pallas_call(
        body,
        grid=grid,
        in_specs=[
            pl.BlockSpec((bn, L), lambda i: (i, 0)),
            pl.BlockSpec((bn, L), lambda i: (i, 0)),
        ],
        out_specs=pl.BlockSpec((bn, L), lambda i: (i, 0)),
        out_shape=jax.ShapeDtypeStruct((n_pad, L), F32),
    )(h_pad, dinv16)


def _tc_layer1(agg1, g1, dinv16, w1p, b1):
    n_pad = g1.shape[0]
    bn = 2048
    grid = (n_pad // bn,)

    def body(a_ref, g_ref, d_ref, w_ref, b_ref, h1_ref, g2_ref):
        d16 = d_ref[...]
        ah = d16 * (a_ref[0] + a_ref[1] + g_ref[...])
        h1 = jnp.dot(ah, w_ref[...], preferred_element_type=F32) + b_ref[...]
        h1 = jnp.maximum(h1, 0.0)
        h1_ref[...] = h1
        g2 = h1 * jnp.concatenate([d16, d16], axis=1)
        g2_ref[0] = g2[:, :L]
        g2_ref[1] = g2[:, L:]

    return pl.pallas_call(
        body,
        grid=grid,
        in_specs=[
            pl.BlockSpec((NC, bn, L), lambda i: (0, i, 0)),
            pl.BlockSpec((bn, L), lambda i: (i, 0)),
            pl.BlockSpec((bn, L), lambda i: (i, 0)),
            pl.BlockSpec((L, 32), lambda i: (0, 0)),
            pl.BlockSpec((1, 32), lambda i: (0, 0)),
        ],
        out_specs=[
            pl.BlockSpec((bn, 32), lambda i: (i, 0)),
            pl.BlockSpec((NC, bn, L), lambda i: (0, i, 0)),
        ],
        out_shape=[
            jax.ShapeDtypeStruct((n_pad, 32), F32),
            jax.ShapeDtypeStruct((NC, n_pad, L), F32),
        ],
    )(agg1, g1, dinv16, w1p, b1)


def _tc_layer23(agg2, h1, dinv16, w2, b2, w3):
    n_pad = h1.shape[0]
    bn = 2048
    grid = (n_pad // bn,)

    def body(a_ref, h1_ref, d_ref, w2_ref, b2_ref, w3_ref, g3_ref):
        d16 = d_ref[...]
        d32 = jnp.concatenate([d16, d16], axis=1)
        s2 = jnp.concatenate([a_ref[0], a_ref[1]], axis=1)
        ah2 = d32 * (s2 + d32 * h1_ref[...])
        h2 = jnp.dot(ah2, w2_ref[...], preferred_element_type=F32) + b2_ref[...]
        h2 = jnp.maximum(h2, 0.0)
        z = jnp.dot(h2, w3_ref[...], preferred_element_type=F32)
        col = lax.broadcasted_iota(jnp.int32, (bn, L), 1)
        g3_ref[...] = jnp.where(col == 0, z * d16, 0.0)

    return pl.pallas_call(
        body,
        grid=grid,
        in_specs=[
            pl.BlockSpec((NC, bn, L), lambda i: (0, i, 0)),
            pl.BlockSpec((bn, 32), lambda i: (i, 0)),
            pl.BlockSpec((bn, L), lambda i: (i, 0)),
            pl.BlockSpec((32, 32), lambda i: (0, 0)),
            pl.BlockSpec((1, 32), lambda i: (0, 0)),
            pl.BlockSpec((32, 1), lambda i: (0, 0)),
        ],
        out_specs=pl.BlockSpec((bn, L), lambda i: (i, 0)),
        out_shape=jax.ShapeDtypeStruct((n_pad, L), F32),
    )(agg2, h1, dinv16, w2, b2, w3)


def _tc_final(agg3, g3, dinv16, b3):
    n_pad = g3.shape[0]
    bn = 2048
    grid = (n_pad // bn,)

    def body(a_ref, g_ref, d_ref, b_ref, o_ref):
        o_ref[...] = d_ref[...] * (a_ref[0] + a_ref[1] + g_ref[...]) + b_ref[0, 0]

    return pl.pallas_call(
        body,
        grid=grid,
        in_specs=[
            pl.BlockSpec((NC, bn, L), lambda i: (0, i, 0)),
            pl.BlockSpec((bn, L), lambda i: (i, 0)),
            pl.BlockSpec((bn, L), lambda i: (i, 0)),
            pl.BlockSpec((1, 1), lambda i: (0, 0)),
        ],
        out_specs=pl.BlockSpec((bn, L), lambda i: (i, 0)),
        out_shape=jax.ShapeDtypeStruct((n_pad, L), F32),
    )(agg3, g3, dinv16, b3)


def kernel(x, y1, edge_index, W1, b1, W2, b2, W3, b3):
    n = x.shape[0]
    e = edge_index.shape[1]

    e_align = NC * NS * GROUP
    e_pad = ((e + e_align - 1) // e_align) * e_align
    n_align = NS * BUFR
    n_pad = ((n + 1 + n_align - 1) // n_align) * n_align

    src = edge_index[0].astype(jnp.int32)
    dst = edge_index[1].astype(jnp.int32)
    src2d = jnp.pad(src, (0, e_pad - e), constant_values=n).reshape(e_pad // CH, CH)
    dst2d = jnp.pad(dst, (0, e_pad - e), constant_values=n).reshape(e_pad // CH, CH)

    deg_parts = _make_deg_kernel(n_pad, e_pad)(dst2d)
    dinv = _tc_dinv(deg_parts.reshape(NC, n_pad // 128, 128))
    dinv16 = jnp.broadcast_to(dinv.reshape(n_pad, 1), (n_pad, L))

    h = jnp.concatenate([x, y1.reshape(-1, 1)], axis=1)
    h_pad = jnp.pad(h, ((0, n_pad - n), (0, L - h.shape[1])))
    g1 = _tc_scale(h_pad, dinv16)
    agg1 = _make_agg_kernel(n_pad, e_pad, 1)(g1, src2d, dst2d)

    w1p = jnp.pad(W1, ((0, L - W1.shape[0]), (0, 0)))
    h1, g2s = _tc_layer1(agg1, g1, dinv16, w1p, b1.reshape(1, 32))

    agg2 = _make_agg_kernel(n_pad, e_pad, 2)(
        g2s.reshape(NC * n_pad, L), src2d, dst2d
    )
    g3 = _tc_layer23(agg2, h1, dinv16, W2, b2.reshape(1, 32), W3)

    agg3 = _make_agg_kernel(n_pad, e_pad, 1)(g3, src2d, dst2d)
    f = _tc_final(agg3, g3, dinv16, b3.reshape(1, 1))
    return f[:n, 0]


# speedup vs baseline: 36.8757x; 1.0867x over previous
import jax
import jax.numpy as jnp
from jax import lax
from jax.experimental import pallas as pl
from jax.experimental.pallas import tpu as pltpu
from jax.experimental.pallas import tpu_sc as plsc

NC = 2
NS = 16
L = 16

CH = 512
F32 = jnp.float32
BN = 2048


def _mesh():
    return plsc.VectorSubcoreMesh(
        core_axis_name="c", subcore_axis_name="s", num_cores=NC, num_subcores=NS
    )


def _make_deg_kernel(n_pad, e_pad):
    e_half = e_pad // NC
    rows_per_tile_e = e_half // NS // CH
    n_groups = rows_per_tile_e
    rpt = n_pad // NS

    def body(dst2d, zer, out, idx_v, ones_v, buf, acc, sem):
        c = lax.axis_index("c")
        s = lax.axis_index("s")
        row0 = s * rpt

        @pl.loop(0, CH // L)
        def _ones(j):
            ones_v[pl.ds(j * L, L)] = jnp.ones((L,), F32)

        pltpu.sync_copy(zer.at[pl.ds(row0, rpt)], buf)
        pltpu.sync_copy(buf, acc.at[pl.ds(row0, rpt)])
        plsc.subcore_barrier()

        tile_erow0 = c * (e_half // CH) + s * rows_per_tile_e

        @pl.loop(0, n_groups)
        def _grp(g):
            pltpu.sync_copy(dst2d.at[tile_erow0 + g], idx_v)
            pltpu.async_copy(ones_v, acc.at[idx_v], sem, add=True).wait()

        plsc.subcore_barrier()
        pltpu.sync_copy(acc.at[pl.ds(row0, rpt)], buf)
        pltpu.sync_copy(buf, out.at[c, pl.ds(row0, rpt)])

    return pl.kernel(
        body,
        out_type=jax.ShapeDtypeStruct((NC, n_pad), F32),
        mesh=_mesh(),
        compiler_params=pltpu.CompilerParams(use_tc_tiling_on_sc=False),
        scratch_types=[
            pltpu.VMEM((CH,), jnp.int32),
            pltpu.VMEM((CH,), F32),
            pltpu.VMEM((rpt,), F32),
            pltpu.VMEM_SHARED((n_pad,), F32),
            pltpu.SemaphoreType.DMA,
        ],
    )


def _make_agg_kernel(n_pad, e_pad, n_tables, w):
    split_edges = n_tables == 1
    e_per_sc = e_pad // NC if split_edges else e_pad
    rows_per_tile_e = e_per_sc // NS // CH
    n_groups = rows_per_tile_e
    rpt = n_pad // NS
    assert n_groups % 2 == 0

    bufr = rpt // 14

    def body(table, src2d, dst2d, zer, out,
             idxs0, idxd0, idxs1, idxd1, rows0, rows1, buf, acc,
             semg0, semg1, sems0, sems1):
        c = lax.axis_index("c")
        s = lax.axis_index("s")
        row0 = s * rpt

        @pl.loop(0, rpt // bufr)
        def _zacc(i):
            pltpu.sync_copy(zer.at[pl.ds(row0 + i * bufr, bufr)], buf)
            pltpu.sync_copy(buf, acc.at[pl.ds(row0 + i * bufr, bufr)])

        plsc.subcore_barrier()

        if split_edges:
            tile_erow0 = c * (e_per_sc // CH) + s * rows_per_tile_e
        else:
            tile_erow0 = s * rows_per_tile_e
        off = None if split_edges else c * n_pad

        def load_idx(er, ixs, ixd):
            pltpu.sync_copy(src2d.at[er], ixs)
            pltpu.sync_copy(dst2d.at[er], ixd)
            if off is not None:
                for t in range(CH // L):
                    ixs[pl.ds(t * L, L)] = ixs[pl.ds(t * L, L)] + off

        load_idx(tile_erow0, idxs0, idxd0)
        pltpu.async_copy(table.at[idxs0], rows0, semg0)

        @pl.loop(0, n_groups // 2)
        def _grp(i):
            g = 2 * i
            load_idx(tile_erow0 + g + 1, idxs1, idxd1)

            @pl.when(i > 0)
            def _():
                pltpu.make_async_copy(rows1, acc.at[idxd1], sems1).wait()

            pltpu.make_async_copy(table.at[idxs0], rows0, semg0).wait()
            pltpu.async_copy(rows0, acc.at[idxd0], sems0, add=True)
            pltpu.async_copy(table.at[idxs1], rows1, semg1)

            pltpu.make_async_copy(rows0, acc.at[idxd0], sems0).wait()

            @pl.when(g + 2 < n_groups)
            def _():
                load_idx(tile_erow0 + g + 2, idxs0, idxd0)
                pltpu.async_copy(table.at[idxs0], rows0, semg0)

            pltpu.make_async_copy(table.at[idxs1], rows1, semg1).wait()
            pltpu.async_copy(rows1, acc.at[idxd1], sems1, add=True)

        pltpu.make_async_copy(rows1, acc.at[idxd1], sems1).wait()
        plsc.subcore_barrier()

        @pl.loop(0, rpt // bufr)
        def _wr(i):
            pltpu.sync_copy(acc.at[pl.ds(row0 + i * bufr, bufr)], buf)
            pltpu.sync_copy(buf, out.at[c, pl.ds(row0 + i * bufr, bufr)])

    return pl.kernel(
        body,
        out_type=jax.ShapeDtypeStruct((NC, n_pad, w), F32),
        mesh=_mesh(),
        compiler_params=pltpu.CompilerParams(use_tc_tiling_on_sc=False),
        scratch_types=[
            pltpu.VMEM((CH,), jnp.int32),
            pltpu.VMEM((CH,), jnp.int32),
            pltpu.VMEM((CH,), jnp.int32),
            pltpu.VMEM((CH,), jnp.int32),
            pltpu.VMEM((CH, w), F32),
            pltpu.VMEM((CH, w), F32),
            pltpu.VMEM((rpt // 14, w), F32),
            pltpu.VMEM_SHARED((n_pad, w), F32),
            pltpu.SemaphoreType.DMA,
            pltpu.SemaphoreType.DMA,
            pltpu.SemaphoreType.DMA,
            pltpu.SemaphoreType.DMA,
        ],
    )


def _tc_dinv(d_parts):
    def body(d_ref, o_ref):
        o_ref[...] = lax.rsqrt(d_ref[0] + d_ref[1] + 1.0)

    r = d_parts.shape[1]
    return pl.pallas_call(
        body,
        out_shape=jax.ShapeDtypeStruct((r, 128), F32),
    )(d_parts)


def _bc(d, w):
    return jnp.broadcast_to(d, (d.shape[0], w))


def _tc_scale(h_pad, dinv_col):
    n_pad, w = h_pad.shape

    def body(h_ref, d_ref, o_ref):
        o_ref[...] = h_ref[...] * _bc(d_ref[...], w)

    return pl.pallas_call(
        body,
        grid=(n_pad // BN,),
        in_specs=[
            pl.BlockSpec((BN, w), lambda i: (i, 0)),
            pl.BlockSpec((BN, 1), lambda i: (i, 0)),
        ],
        out_specs=pl.BlockSpec((BN, w), lambda i: (i, 0)),
        out_shape=jax.ShapeDtypeStruct((n_pad, w), F32),
    )(h_pad, dinv_col)


def _tc_layer1(agg1, g1, dinv_col, w1p, b1):
    n_pad, w = g1.shape

    def body(a_ref, g_ref, d_ref, w_ref, b_ref, g2_ref):
        d = d_ref[...]
        ah = _bc(d, w) * (a_ref[0] + a_ref[1] + g_ref[...])
        h1 = jnp.dot(ah, w_ref[...], preferred_element_type=F32) + b_ref[...]
        g2 = jnp.maximum(h1, 0.0) * _bc(d, 32)
        g2_ref[0] = g2[:, :L]
        g2_ref[1] = g2[:, L:]

    return pl.pallas_call(
        body,
        grid=(n_pad // BN,),
        in_specs=[
            pl.BlockSpec((NC, BN, w), lambda i: (0, i, 0)),
            pl.BlockSpec((BN, w), lambda i: (i, 0)),
            pl.BlockSpec((BN, 1), lambda i: (i, 0)),
            pl.BlockSpec((w, 32), lambda i: (0, 0)),
            pl.BlockSpec((1, 32), lambda i: (0, 0)),
        ],
        out_specs=pl.BlockSpec((NC, BN, L), lambda i: (0, i, 0)),
        out_shape=jax.ShapeDtypeStruct((NC, n_pad, L), F32),
    )(agg1, g1, dinv_col, w1p, b1)


def _tc_layer23(agg2, g2s, dinv_col, w2, b2, w3, w_out):
    n_pad = agg2.shape[1]

    def body(a_ref, g2_ref, d_ref, w2_ref, b2_ref, w3_ref, g3_ref):
        d = d_ref[...]
        s2 = jnp.concatenate([a_ref[0], a_ref[1]], axis=1)
        g2 = jnp.concatenate([g2_ref[0], g2_ref[1]], axis=1)
        ah2 = _bc(d, 32) * (s2 + g2)
        h2 = jnp.dot(ah2, w2_ref[...], preferred_element_type=F32) + b2_ref[...]
        h2 = jnp.maximum(h2, 0.0)
        z = jnp.dot(h2, w3_ref[...], preferred_element_type=F32)
        col = lax.broadcasted_iota(jnp.int32, (BN, w_out), 1)
        g3_ref[...] = jnp.where(col == 0, z * d, 0.0)

    return pl.pallas_call(
        body,
        grid=(n_pad // BN,),
        in_specs=[
            pl.BlockSpec((NC, BN, L), lambda i: (0, i, 0)),
            pl.BlockSpec((NC, BN, L), lambda i: (0, i, 0)),
            pl.BlockSpec((BN, 1), lambda i: (i, 0)),
            pl.BlockSpec((32, 32), lambda i: (0, 0)),
            pl.BlockSpec((1, 32), lambda i: (0, 0)),
            pl.BlockSpec((32, 1), lambda i: (0, 0)),
        ],
        out_specs=pl.BlockSpec((BN, w_out), lambda i: (i, 0)),
        out_shape=jax.ShapeDtypeStruct((n_pad, w_out), F32),
    )(agg2, g2s, dinv_col, w2, b2, w3)


def _tc_final(agg3, g3, dinv_col, b3):
    n_pad, w = g3.shape

    def body(a_ref, g_ref, d_ref, b_ref, o_ref):
        f = d_ref[...] * (a_ref[0, :, :1] + a_ref[1, :, :1] + g_ref[:, :1])
        o_ref[...] = f + b_ref[0, 0]

    return pl.pallas_call(
        body,
        grid=(n_pad // BN,),
        in_specs=[
            pl.BlockSpec((NC, BN, w), lambda i: (0, i, 0)),
            pl.BlockSpec((BN, w), lambda i: (i, 0)),
            pl.BlockSpec((BN, 1), lambda i: (i, 0)),
            pl.BlockSpec((1, 1), lambda i: (0, 0)),
        ],
        out_specs=pl.BlockSpec((BN, 1), lambda i: (i, 0)),
        out_shape=jax.ShapeDtypeStruct((n_pad, 1), F32),
    )(agg3, g3, dinv_col, b3)


def kernel(x, y1, edge_index, W1, b1, W2, b2, W3, b3):
    n = x.shape[0]
    e = edge_index.shape[1]
    w_small = 16

    e_align = NC * NS * CH * 2
    e_pad = ((e + e_align - 1) // e_align) * e_align
    n_align = BN
    n_pad = ((n + 1 + n_align - 1) // n_align) * n_align

    src = edge_index[0].astype(jnp.int32)
    dst = edge_index[1].astype(jnp.int32)
    src2d = jnp.pad(src, (0, e_pad - e), constant_values=n).reshape(e_pad // CH, CH)
    dst2d = jnp.pad(dst, (0, e_pad - e), constant_values=n).reshape(e_pad // CH, CH)

    zeros1 = jnp.zeros((n_pad,), F32)
    zeros4 = jnp.zeros((n_pad, w_small), F32)
    zeros16 = jnp.zeros((n_pad, L), F32)

    deg_parts = _make_deg_kernel(n_pad, e_pad)(dst2d, zeros1)
    dinv = _tc_dinv(deg_parts.reshape(NC, n_pad // 128, 128))
    dinv_col = dinv.reshape(n_pad, 1)

    h = jnp.concatenate([x, y1.reshape(-1, 1)], axis=1)
    h_pad = jnp.pad(h, ((0, n_pad - n), (0, w_small - h.shape[1])))
    g1 = _tc_scale(h_pad, dinv_col)
    agg1 = _make_agg_kernel(n_pad, e_pad, 1, w_small)(g1, src2d, dst2d, zeros4)

    w1p = jnp.pad(W1, ((0, w_small - W1.shape[0]), (0, 0)))
    g2s = _tc_layer1(agg1, g1, dinv_col, w1p, b1.reshape(1, 32))

    agg2 = _make_agg_kernel(n_pad, e_pad, 2, L)(
        g2s.reshape(NC * n_pad, L), src2d, dst2d, zeros16
    )
    g3 = _tc_layer23(agg2, g2s, dinv_col, W2, b2.reshape(1, 32), W3, w_small)

    agg3 = _make_agg_kernel(n_pad, e_pad, 1, w_small)(g3, src2d, dst2d, zeros4)
    f = _tc_final(agg3, g3, dinv_col, b3.reshape(1, 1))
    return f[:n, 0]


# speedup vs baseline: 37.5920x; 1.0194x over previous
import jax
import jax.numpy as jnp
from jax import lax
from jax.experimental import pallas as pl
from jax.experimental.pallas import tpu as pltpu
from jax.experimental.pallas import tpu_sc as plsc

NC = 2
NS = 16
L = 16

CH = 512
F32 = jnp.float32
BN = 2048


def _mesh():
    return plsc.VectorSubcoreMesh(
        core_axis_name="c", subcore_axis_name="s", num_cores=NC, num_subcores=NS
    )


def _make_deg_kernel(n_pad, e_pad):
    e_half = e_pad // NC
    rows_per_tile_e = e_half // NS // CH
    n_groups = rows_per_tile_e
    rpt = n_pad // NS

    def body(dst2d, out, idx_v, ones_v, buf, acc, sem):
        c = lax.axis_index("c")
        s = lax.axis_index("s")
        row0 = s * rpt

        @pl.loop(0, CH // L)
        def _ones(j):
            ones_v[pl.ds(j * L, L)] = jnp.ones((L,), F32)

        @pl.loop(0, rpt // L)
        def _zero(i):
            buf[pl.ds(i * L, L)] = jnp.zeros((L,), F32)

        pltpu.sync_copy(buf, acc.at[pl.ds(row0, rpt)])
        plsc.subcore_barrier()

        tile_erow0 = c * (e_half // CH) + s * rows_per_tile_e

        @pl.loop(0, n_groups)
        def _grp(g):
            pltpu.sync_copy(dst2d.at[tile_erow0 + g], idx_v)
            pltpu.async_copy(ones_v, acc.at[idx_v], sem, add=True).wait()

        plsc.subcore_barrier()
        pltpu.sync_copy(acc.at[pl.ds(row0, rpt)], buf)
        pltpu.sync_copy(buf, out.at[c, pl.ds(row0, rpt)])

    return pl.kernel(
        body,
        out_type=jax.ShapeDtypeStruct((NC, n_pad), F32),
        mesh=_mesh(),
        compiler_params=pltpu.CompilerParams(use_tc_tiling_on_sc=False),
        scratch_types=[
            pltpu.VMEM((CH,), jnp.int32),
            pltpu.VMEM((CH,), F32),
            pltpu.VMEM((rpt,), F32),
            pltpu.VMEM_SHARED((n_pad,), F32),
            pltpu.SemaphoreType.DMA,
        ],
    )


def _make_agg_kernel(n_pad, e_pad, n_tables, w):
    split_edges = n_tables == 1
    e_per_sc = e_pad // NC if split_edges else e_pad
    rows_per_tile_e = e_per_sc // NS // CH
    n_groups = rows_per_tile_e
    rpt = n_pad // NS
    assert n_groups % 2 == 0

    bufr = rpt // 14
    assert w == L

    def body(table, src2d, dst2d, out,
             idxs0, idxd0, idxs1, idxd1, rows0, rows1, buf, acc,
             semg0, semg1, sems0, sems1):
        c = lax.axis_index("c")
        s = lax.axis_index("s")
        row0 = s * rpt

        @pl.loop(0, bufr)
        def _zf(i):
            buf[i, :] = jnp.zeros((L,), F32)

        @pl.loop(0, rpt // bufr)
        def _zacc(i):
            pltpu.sync_copy(buf, acc.at[pl.ds(row0 + i * bufr, bufr)])

        plsc.subcore_barrier()

        if split_edges:
            tile_erow0 = c * (e_per_sc // CH) + s * rows_per_tile_e
        else:
            tile_erow0 = s * rows_per_tile_e
        off = None if split_edges else c * n_pad

        def load_idx(er, ixs, ixd):
            pltpu.sync_copy(src2d.at[er], ixs)
            pltpu.sync_copy(dst2d.at[er], ixd)
            if off is not None:
                for t in range(CH // L):
                    ixs[pl.ds(t * L, L)] = ixs[pl.ds(t * L, L)] + off

        load_idx(tile_erow0, idxs0, idxd0)
        pltpu.async_copy(table.at[idxs0], rows0, semg0)

        @pl.loop(0, n_groups // 2)
        def _grp(i):
            g = 2 * i
            load_idx(tile_erow0 + g + 1, idxs1, idxd1)

            @pl.when(i > 0)
            def _():
                pltpu.make_async_copy(rows1, acc.at[idxd1], sems1).wait()

            pltpu.make_async_copy(table.at[idxs0], rows0, semg0).wait()
            pltpu.async_copy(rows0, acc.at[idxd0], sems0, add=True)
            pltpu.async_copy(table.at[idxs1], rows1, semg1)

            pltpu.make_async_copy(rows0, acc.at[idxd0], sems0).wait()

            @pl.when(g + 2 < n_groups)
            def _():
                load_idx(tile_erow0 + g + 2, idxs0, idxd0)
                pltpu.async_copy(table.at[idxs0], rows0, semg0)

            pltpu.make_async_copy(table.at[idxs1], rows1, semg1).wait()
            pltpu.async_copy(rows1, acc.at[idxd1], sems1, add=True)

        pltpu.make_async_copy(rows1, acc.at[idxd1], sems1).wait()
        plsc.subcore_barrier()

        @pl.loop(0, rpt // bufr)
        def _wr(i):
            pltpu.sync_copy(acc.at[pl.ds(row0 + i * bufr, bufr)], buf)
            pltpu.sync_copy(buf, out.at[c, pl.ds(row0 + i * bufr, bufr)])

    return pl.kernel(
        body,
        out_type=jax.ShapeDtypeStruct((NC, n_pad, w), F32),
        mesh=_mesh(),
        compiler_params=pltpu.CompilerParams(use_tc_tiling_on_sc=False),
        scratch_types=[
            pltpu.VMEM((CH,), jnp.int32),
            pltpu.VMEM((CH,), jnp.int32),
            pltpu.VMEM((CH,), jnp.int32),
            pltpu.VMEM((CH,), jnp.int32),
            pltpu.VMEM((CH, w), F32),
            pltpu.VMEM((CH, w), F32),
            pltpu.VMEM((rpt // 14, w), F32),
            pltpu.VMEM_SHARED((n_pad, w), F32),
            pltpu.SemaphoreType.DMA,
            pltpu.SemaphoreType.DMA,
            pltpu.SemaphoreType.DMA,
            pltpu.SemaphoreType.DMA,
        ],
    )


def _tc_dinv(d_parts):
    def body(d_ref, o_ref):
        o_ref[...] = lax.rsqrt(d_ref[0] + d_ref[1] + 1.0)

    r = d_parts.shape[1]
    return pl.pallas_call(
        body,
        out_shape=jax.ShapeDtypeStruct((r, 128), F32),
    )(d_parts)


def _bc(d, w):
    return jnp.broadcast_to(d, (d.shape[0], w))


def _tc_scale(h_pad, dinv_col):
    n_pad, w = h_pad.shape

    def body(h_ref, d_ref, o_ref):
        o_ref[...] = h_ref[...] * _bc(d_ref[...], w)

    return pl.pallas_call(
        body,
        grid=(n_pad // BN,),
        in_specs=[
            pl.BlockSpec((BN, w), lambda i: (i, 0)),
            pl.BlockSpec((BN, 1), lambda i: (i, 0)),
        ],
        out_specs=pl.BlockSpec((BN, w), lambda i: (i, 0)),
        out_shape=jax.ShapeDtypeStruct((n_pad, w), F32),
    )(h_pad, dinv_col)


def _tc_layer1(agg1, g1, dinv_col, w1p, b1):
    n_pad, w = g1.shape

    def body(a_ref, g_ref, d_ref, w_ref, b_ref, g2_ref):
        d = d_ref[...]
        ah = _bc(d, w) * (a_ref[0] + a_ref[1] + g_ref[...])
        h1 = jnp.dot(ah, w_ref[...], preferred_element_type=F32) + b_ref[...]
        g2 = jnp.maximum(h1, 0.0) * _bc(d, 32)
        g2_ref[0] = g2[:, :L]
        g2_ref[1] = g2[:, L:]

    return pl.pallas_call(
        body,
        grid=(n_pad // BN,),
        in_specs=[
            pl.BlockSpec((NC, BN, w), lambda i: (0, i, 0)),
            pl.BlockSpec((BN, w), lambda i: (i, 0)),
            pl.BlockSpec((BN, 1), lambda i: (i, 0)),
            pl.BlockSpec((w, 32), lambda i: (0, 0)),
            pl.BlockSpec((1, 32), lambda i: (0, 0)),
        ],
        out_specs=pl.BlockSpec((NC, BN, L), lambda i: (0, i, 0)),
        out_shape=jax.ShapeDtypeStruct((NC, n_pad, L), F32),
    )(agg1, g1, dinv_col, w1p, b1)


def _tc_layer23(agg2, g2s, dinv_col, w2, b2, w3, w_out):
    n_pad = agg2.shape[1]

    def body(a_ref, g2_ref, d_ref, w2_ref, b2_ref, w3_ref, g3_ref):
        d = d_ref[...]
        s2 = jnp.concatenate([a_ref[0], a_ref[1]], axis=1)
        g2 = jnp.concatenate([g2_ref[0], g2_ref[1]], axis=1)
        ah2 = _bc(d, 32) * (s2 + g2)
        h2 = jnp.dot(ah2, w2_ref[...], preferred_element_type=F32) + b2_ref[...]
        h2 = jnp.maximum(h2, 0.0)
        z = jnp.dot(h2, w3_ref[...], preferred_element_type=F32)
        col = lax.broadcasted_iota(jnp.int32, (BN, w_out), 1)
        g3_ref[...] = jnp.where(col == 0, z * d, 0.0)

    return pl.pallas_call(
        body,
        grid=(n_pad // BN,),
        in_specs=[
            pl.BlockSpec((NC, BN, L), lambda i: (0, i, 0)),
            pl.BlockSpec((NC, BN, L), lambda i: (0, i, 0)),
            pl.BlockSpec((BN, 1), lambda i: (i, 0)),
            pl.BlockSpec((32, 32), lambda i: (0, 0)),
            pl.BlockSpec((1, 32), lambda i: (0, 0)),
            pl.BlockSpec((32, 1), lambda i: (0, 0)),
        ],
        out_specs=pl.BlockSpec((BN, w_out), lambda i: (i, 0)),
        out_shape=jax.ShapeDtypeStruct((n_pad, w_out), F32),
    )(agg2, g2s, dinv_col, w2, b2, w3)


def _tc_final(agg3, g3, dinv_col, b3):
    n_pad, w = g3.shape

    def body(a_ref, g_ref, d_ref, b_ref, o_ref):
        f = d_ref[...] * (a_ref[0, :, :1] + a_ref[1, :, :1] + g_ref[:, :1])
        o_ref[...] = f + b_ref[0, 0]

    return pl.pallas_call(
        body,
        grid=(n_pad // BN,),
        in_specs=[
            pl.BlockSpec((NC, BN, w), lambda i: (0, i, 0)),
            pl.BlockSpec((BN, w), lambda i: (i, 0)),
            pl.BlockSpec((BN, 1), lambda i: (i, 0)),
            pl.BlockSpec((1, 1), lambda i: (0, 0)),
        ],
        out_specs=pl.BlockSpec((BN, 1), lambda i: (i, 0)),
        out_shape=jax.ShapeDtypeStruct((n_pad, 1), F32),
    )(agg3, g3, dinv_col, b3)


def kernel(x, y1, edge_index, W1, b1, W2, b2, W3, b3):
    n = x.shape[0]
    e = edge_index.shape[1]
    w_small = 16

    e_align = NC * NS * CH * 2
    e_pad = ((e + e_align - 1) // e_align) * e_align
    n_align = BN
    n_pad = ((n + 1 + n_align - 1) // n_align) * n_align

    src = edge_index[0].astype(jnp.int32)
    dst = edge_index[1].astype(jnp.int32)
    src2d = jnp.pad(src, (0, e_pad - e), constant_values=n).reshape(e_pad // CH, CH)
    dst2d = jnp.pad(dst, (0, e_pad - e), constant_values=n).reshape(e_pad // CH, CH)

    deg_parts = _make_deg_kernel(n_pad, e_pad)(dst2d)
    dinv = _tc_dinv(deg_parts.reshape(NC, n_pad // 128, 128))
    dinv_col = dinv.reshape(n_pad, 1)

    h = jnp.concatenate([x, y1.reshape(-1, 1)], axis=1)
    h_pad = jnp.pad(h, ((0, n_pad - n), (0, w_small - h.shape[1])))
    g1 = _tc_scale(h_pad, dinv_col)
    agg1 = _make_agg_kernel(n_pad, e_pad, 1, w_small)(g1, src2d, dst2d)

    w1p = jnp.pad(W1, ((0, w_small - W1.shape[0]), (0, 0)))
    g2s = _tc_layer1(agg1, g1, dinv_col, w1p, b1.reshape(1, 32))

    agg2 = _make_agg_kernel(n_pad, e_pad, 2, L)(
        g2s.reshape(NC * n_pad, L), src2d, dst2d
    )
    g3 = _tc_layer23(agg2, g2s, dinv_col, W2, b2.reshape(1, 32), W3, w_small)

    agg3 = _make_agg_kernel(n_pad, e_pad, 1, w_small)(g3, src2d, dst2d)
    f = _tc_final(agg3, g3, dinv_col, b3.reshape(1, 1))
    return f[:n, 0]


# speedup vs baseline: 37.9583x; 1.0097x over previous
import jax
import jax.numpy as jnp
from jax import lax
from jax.experimental import pallas as pl
from jax.experimental.pallas import tpu as pltpu
from jax.experimental.pallas import tpu_sc as plsc

NC = 2
NS = 16
L = 16

CH = 512
F32 = jnp.float32
BN = 2048


def _mesh():
    return plsc.VectorSubcoreMesh(
        core_axis_name="c", subcore_axis_name="s", num_cores=NC, num_subcores=NS
    )


def _make_deg_kernel(n_pad, e_pad):
    e_half = e_pad // NC
    rows_per_tile_e = e_half // NS // CH
    n_groups = rows_per_tile_e
    rpt = n_pad // NS

    def body(dst2d, out, idx_v, ones_v, buf, acc, sem):
        c = lax.axis_index("c")
        s = lax.axis_index("s")
        row0 = s * rpt

        @pl.loop(0, CH // L)
        def _ones(j):
            ones_v[pl.ds(j * L, L)] = jnp.ones((L,), F32)

        @pl.loop(0, rpt // L)
        def _zero(i):
            buf[pl.ds(i * L, L)] = jnp.zeros((L,), F32)

        pltpu.sync_copy(buf, acc.at[pl.ds(row0, rpt)])
        plsc.subcore_barrier()

        tile_erow0 = c * (e_half // CH) + s * rows_per_tile_e

        @pl.loop(0, n_groups)
        def _grp(g):
            pltpu.sync_copy(dst2d.at[tile_erow0 + g], idx_v)
            pltpu.async_copy(ones_v, acc.at[idx_v], sem, add=True).wait()

        plsc.subcore_barrier()
        pltpu.sync_copy(acc.at[pl.ds(row0, rpt)], buf)
        pltpu.sync_copy(buf, out.at[c, pl.ds(row0, rpt)])

    return pl.kernel(
        body,
        out_type=jax.ShapeDtypeStruct((NC, n_pad), F32),
        mesh=_mesh(),
        compiler_params=pltpu.CompilerParams(use_tc_tiling_on_sc=False),
        scratch_types=[
            pltpu.VMEM((CH,), jnp.int32),
            pltpu.VMEM((CH,), F32),
            pltpu.VMEM((rpt,), F32),
            pltpu.VMEM_SHARED((n_pad,), F32),
            pltpu.SemaphoreType.DMA,
        ],
    )


def _make_agg_kernel(n_pad, e_pad, n_tables, w):
    split_edges = n_tables == 1
    e_per_sc = e_pad // NC if split_edges else e_pad
    rows_per_tile_e = e_per_sc // NS // CH
    n_groups = rows_per_tile_e
    rpt = n_pad // NS
    assert n_groups % 2 == 0

    bufr = rpt // 14

    def body(table, src2d, dst2d, *rest):
        if w == L:
            (out, idxs0, idxd0, idxs1, idxd1, rows0, rows1, buf, acc,
             semg0, semg1, sems0, sems1) = rest
        else:
            (zer, out, idxs0, idxd0, idxs1, idxd1, rows0, rows1, buf, acc,
             semg0, semg1, sems0, sems1) = rest
        c = lax.axis_index("c")
        s = lax.axis_index("s")
        row0 = s * rpt

        if w == L:
            @pl.loop(0, bufr)
            def _zf(i):
                buf[i, :] = jnp.zeros((L,), F32)

            @pl.loop(0, rpt // bufr)
            def _zacc(i):
                pltpu.sync_copy(buf, acc.at[pl.ds(row0 + i * bufr, bufr)])
        else:
            @pl.loop(0, rpt // bufr)
            def _zacc(i):
                pltpu.sync_copy(zer.at[pl.ds(row0 + i * bufr, bufr)], buf)
                pltpu.sync_copy(buf, acc.at[pl.ds(row0 + i * bufr, bufr)])

        plsc.subcore_barrier()

        if split_edges:
            tile_erow0 = c * (e_per_sc // CH) + s * rows_per_tile_e
        else:
            tile_erow0 = s * rows_per_tile_e
        off = None if split_edges else c * n_pad

        def load_idx(er, ixs, ixd):
            pltpu.sync_copy(src2d.at[er], ixs)
            pltpu.sync_copy(dst2d.at[er], ixd)
            if off is not None:
                for t in range(CH // L):
                    ixs[pl.ds(t * L, L)] = ixs[pl.ds(t * L, L)] + off

        load_idx(tile_erow0, idxs0, idxd0)
        pltpu.async_copy(table.at[idxs0], rows0, semg0)

        @pl.loop(0, n_groups // 2)
        def _grp(i):
            g = 2 * i
            load_idx(tile_erow0 + g + 1, idxs1, idxd1)

            @pl.when(i > 0)
            def _():
                pltpu.make_async_copy(rows1, acc.at[idxd1], sems1).wait()

            pltpu.make_async_copy(table.at[idxs0], rows0, semg0).wait()
            pltpu.async_copy(rows0, acc.at[idxd0], sems0, add=True)
            pltpu.async_copy(table.at[idxs1], rows1, semg1)

            pltpu.make_async_copy(rows0, acc.at[idxd0], sems0).wait()

            @pl.when(g + 2 < n_groups)
            def _():
                load_idx(tile_erow0 + g + 2, idxs0, idxd0)
                pltpu.async_copy(table.at[idxs0], rows0, semg0)

            pltpu.make_async_copy(table.at[idxs1], rows1, semg1).wait()
            pltpu.async_copy(rows1, acc.at[idxd1], sems1, add=True)

        pltpu.make_async_copy(rows1, acc.at[idxd1], sems1).wait()
        plsc.subcore_barrier()

        @pl.loop(0, rpt // bufr)
        def _wr(i):
            pltpu.sync_copy(acc.at[pl.ds(row0 + i * bufr, bufr)], buf)
            pltpu.sync_copy(buf, out.at[c, pl.ds(row0 + i * bufr, bufr)])

    return pl.kernel(
        body,
        out_type=jax.ShapeDtypeStruct((NC, n_pad, w), F32),
        mesh=_mesh(),
        compiler_params=pltpu.CompilerParams(use_tc_tiling_on_sc=False),
        scratch_types=[
            pltpu.VMEM((CH,), jnp.int32),
            pltpu.VMEM((CH,), jnp.int32),
            pltpu.VMEM((CH,), jnp.int32),
            pltpu.VMEM((CH,), jnp.int32),
            pltpu.VMEM((CH, w), F32),
            pltpu.VMEM((CH, w), F32),
            pltpu.VMEM((rpt // 14, w), F32),
            pltpu.VMEM_SHARED((n_pad, w), F32),
            pltpu.SemaphoreType.DMA,
            pltpu.SemaphoreType.DMA,
            pltpu.SemaphoreType.DMA,
            pltpu.SemaphoreType.DMA,
        ],
    )


def _tc_dinv(d_parts):
    def body(d_ref, o_ref):
        o_ref[...] = lax.rsqrt(d_ref[0] + d_ref[1] + 1.0)

    r = d_parts.shape[1]
    return pl.pallas_call(
        body,
        out_shape=jax.ShapeDtypeStruct((r, 128), F32),
    )(d_parts)


def _bc(d, w):
    return jnp.broadcast_to(d, (d.shape[0], w))


def _tc_scale(h_pad, dinv_col):
    n_pad, w = h_pad.shape

    def body(h_ref, d_ref, o_ref):
        o_ref[...] = h_ref[...] * _bc(d_ref[...], w)

    return pl.pallas_call(
        body,
        grid=(n_pad // BN,),
        in_specs=[
            pl.BlockSpec((BN, w), lambda i: (i, 0)),
            pl.BlockSpec((BN, 1), lambda i: (i, 0)),
        ],
        out_specs=pl.BlockSpec((BN, w), lambda i: (i, 0)),
        out_shape=jax.ShapeDtypeStruct((n_pad, w), F32),
    )(h_pad, dinv_col)


def _tc_layer1(agg1, g1, dinv_col, w1p, b1):
    n_pad, w = g1.shape

    def body(a_ref, g_ref, d_ref, w_ref, b_ref, g2_ref):
        d = d_ref[...]
        ah = _bc(d, w) * (a_ref[0] + a_ref[1] + g_ref[...])
        h1 = jnp.dot(ah, w_ref[...], preferred_element_type=F32) + b_ref[...]
        g2 = jnp.maximum(h1, 0.0) * _bc(d, 32)
        g2_ref[0] = g2[:, :L]
        g2_ref[1] = g2[:, L:]

    return pl.pallas_call(
        body,
        grid=(n_pad // BN,),
        in_specs=[
            pl.BlockSpec((NC, BN, w), lambda i: (0, i, 0)),
            pl.BlockSpec((BN, w), lambda i: (i, 0)),
            pl.BlockSpec((BN, 1), lambda i: (i, 0)),
            pl.BlockSpec((w, 32), lambda i: (0, 0)),
            pl.BlockSpec((1, 32), lambda i: (0, 0)),
        ],
        out_specs=pl.BlockSpec((NC, BN, L), lambda i: (0, i, 0)),
        out_shape=jax.ShapeDtypeStruct((NC, n_pad, L), F32),
    )(agg1, g1, dinv_col, w1p, b1)


def _tc_layer23(agg2, g2s, dinv_col, w2, b2, w3, w_out):
    n_pad = agg2.shape[1]

    def body(a_ref, g2_ref, d_ref, w2_ref, b2_ref, w3_ref, g3_ref):
        d = d_ref[...]
        s2 = jnp.concatenate([a_ref[0], a_ref[1]], axis=1)
        g2 = jnp.concatenate([g2_ref[0], g2_ref[1]], axis=1)
        ah2 = _bc(d, 32) * (s2 + g2)
        h2 = jnp.dot(ah2, w2_ref[...], preferred_element_type=F32) + b2_ref[...]
        h2 = jnp.maximum(h2, 0.0)
        z = jnp.dot(h2, w3_ref[...], preferred_element_type=F32)
        col = lax.broadcasted_iota(jnp.int32, (BN, w_out), 1)
        g3_ref[...] = jnp.where(col == 0, z * d, 0.0)

    return pl.pallas_call(
        body,
        grid=(n_pad // BN,),
        in_specs=[
            pl.BlockSpec((NC, BN, L), lambda i: (0, i, 0)),
            pl.BlockSpec((NC, BN, L), lambda i: (0, i, 0)),
            pl.BlockSpec((BN, 1), lambda i: (i, 0)),
            pl.BlockSpec((32, 32), lambda i: (0, 0)),
            pl.BlockSpec((1, 32), lambda i: (0, 0)),
            pl.BlockSpec((32, 1), lambda i: (0, 0)),
        ],
        out_specs=pl.BlockSpec((BN, w_out), lambda i: (i, 0)),
        out_shape=jax.ShapeDtypeStruct((n_pad, w_out), F32),
    )(agg2, g2s, dinv_col, w2, b2, w3)


def _tc_final(agg3, g3, dinv_col, b3):
    n_pad, w = g3.shape

    def body(a_ref, g_ref, d_ref, b_ref, o_ref):
        f = d_ref[...] * (a_ref[0, :, :1] + a_ref[1, :, :1] + g_ref[:, :1])
        o_ref[...] = f + b_ref[0, 0]

    return pl.pallas_call(
        body,
        grid=(n_pad // BN,),
        in_specs=[
            pl.BlockSpec((NC, BN, w), lambda i: (0, i, 0)),
            pl.BlockSpec((BN, w), lambda i: (i, 0)),
            pl.BlockSpec((BN, 1), lambda i: (i, 0)),
            pl.BlockSpec((1, 1), lambda i: (0, 0)),
        ],
        out_specs=pl.BlockSpec((BN, 1), lambda i: (i, 0)),
        out_shape=jax.ShapeDtypeStruct((n_pad, 1), F32),
    )(agg3, g3, dinv_col, b3)


def kernel(x, y1, edge_index, W1, b1, W2, b2, W3, b3):
    n = x.shape[0]
    e = edge_index.shape[1]
    w_small = 8

    e_align = NC * NS * CH * 2
    e_pad = ((e + e_align - 1) // e_align) * e_align
    n_align = BN
    n_pad = ((n + 1 + n_align - 1) // n_align) * n_align

    src = edge_index[0].astype(jnp.int32)
    dst = edge_index[1].astype(jnp.int32)
    src2d = jnp.pad(src, (0, e_pad - e), constant_values=n).reshape(e_pad // CH, CH)
    dst2d = jnp.pad(dst, (0, e_pad - e), constant_values=n).reshape(e_pad // CH, CH)

    zeros_w = jnp.zeros((n_pad, w_small), F32)

    deg_parts = _make_deg_kernel(n_pad, e_pad)(dst2d)
    dinv = _tc_dinv(deg_parts.reshape(NC, n_pad // 128, 128))
    dinv_col = dinv.reshape(n_pad, 1)

    h = jnp.concatenate([x, y1.reshape(-1, 1)], axis=1)
    h_pad = jnp.pad(h, ((0, n_pad - n), (0, w_small - h.shape[1])))
    g1 = _tc_scale(h_pad, dinv_col)
    agg1 = _make_agg_kernel(n_pad, e_pad, 1, w_small)(g1, src2d, dst2d, zeros_w)

    w1p = jnp.pad(W1, ((0, w_small - W1.shape[0]), (0, 0)))
    g2s = _tc_layer1(agg1, g1, dinv_col, w1p, b1.reshape(1, 32))

    agg2 = _make_agg_kernel(n_pad, e_pad, 2, L)(
        g2s.reshape(NC * n_pad, L), src2d, dst2d
    )
    g3 = _tc_layer23(agg2, g2s, dinv_col, W2, b2.reshape(1, 32), W3, w_small)

    agg3 = _make_agg_kernel(n_pad, e_pad, 1, w_small)(g3, src2d, dst2d, zeros_w)
    f = _tc_final(agg3, g3, dinv_col, b3.reshape(1, 1))
    return f[:n, 0]
